# Initial kernel scaffold; baseline (speedup 1.0000x reference)
import jax
import jax.numpy as jnp
from jax.experimental import pallas as pl


def kernel(x, edge_index, edge_weight, county_emb, Wx, bx, Wh, bh, W1, b1, W2, b2):
    raise NotImplementedError("write your pallas kernel here")


# speedup vs baseline: 6.9721x; 6.9721x over previous
import functools

import jax
import jax.numpy as jnp
from jax import lax
from jax.experimental import pallas as pl
from jax.experimental.pallas import tpu as pltpu
from jax.experimental.pallas import tpu_sc as plsc

N = 50000
E = 800000
T = 4
F_DIM = 15
EMB = 64

NC = 2
NS = 16
LANES = 16

N_PAD = 50176
ROWS_PT = N_PAD // NS
E_PAD = 819200
CH = 128
NCH32 = E_PAD // (32 * CH)
NCH16 = E_PAD // (16 * CH)
QCH = NCH16 // 5
ZROWS = ROWS_PT // 8

BLK = 512
GRID_N = N_PAD // BLK


def _mesh():
    return plsc.VectorSubcoreMesh(core_axis_name="c", subcore_axis_name="s")


_SC_PARAMS = pltpu.CompilerParams(use_tc_tiling_on_sc=False)


def _wait(src, dst, sem):
    pltpu.make_async_copy(src, dst, sem).wait()


def _sc_deg(src_hbm, w_hbm, out_hbm, idx_v, w_v, zb, acc):
    c = lax.axis_index("c")
    s = lax.axis_index("s")
    wid = c * NS + s

    def zbody(i, _):
        zb[pl.ds(i * LANES, LANES)] = jnp.zeros((LANES,), jnp.float32)
        return 0

    lax.fori_loop(0, ROWS_PT // LANES, zbody, 0)
    pltpu.sync_copy(zb, acc.at[pl.ds(s * ROWS_PT, ROWS_PT)])
    plsc.subcore_barrier()

    pltpu.sync_copy(src_hbm.at[wid], idx_v)
    pltpu.sync_copy(w_hbm.at[wid], w_v)

    def body(ci, _):
        pltpu.sync_copy(w_v.at[ci], acc.at[idx_v.at[ci]], add=True)
        return 0

    lax.fori_loop(0, NCH32, body, 0)
    plsc.subcore_barrier()
    pltpu.sync_copy(acc.at[pl.ds(s * ROWS_PT, ROWS_PT)], zb)
    pltpu.sync_copy(zb, out_hbm.at[pl.ds(c * N_PAD + s * ROWS_PT, ROWS_PT)])


def _deg_call(src32, w32):
    kfn = pl.kernel(
        _sc_deg,
        out_type=jax.ShapeDtypeStruct((NC * N_PAD,), jnp.float32),
        mesh=_mesh(),
        compiler_params=_SC_PARAMS,
        scratch_types=[
            pltpu.VMEM((NCH32, CH), jnp.int32),
            pltpu.VMEM((NCH32, CH), jnp.float32),
            pltpu.VMEM((ROWS_PT,), jnp.float32),
            pltpu.VMEM_SHARED((N_PAD,), jnp.float32),
        ],
    )
    return kfn(src32, w32)


def _sc_norm(src_hbm, dst_hbm, w_hbm, dinv_hbm, out_hbm,
             s_q, d_q, w_q, n_q, ds_b, dd_b, gsem):
    c = lax.axis_index("c")
    s = lax.axis_index("s")
    wid = c * NS + s
    nq = NCH32 // 5

    for q in range(5):
        pltpu.sync_copy(src_hbm.at[wid, pl.ds(q * nq, nq)], s_q)
        pltpu.sync_copy(dst_hbm.at[wid, pl.ds(q * nq, nq)], d_q)
        pltpu.sync_copy(w_hbm.at[wid, pl.ds(q * nq, nq)], w_q)

        def fire(ci, _):
            pltpu.async_copy(dinv_hbm.at[s_q.at[ci]], ds_b.at[ci], gsem)
            pltpu.async_copy(dinv_hbm.at[d_q.at[ci]], dd_b.at[ci], gsem)
            return 0

        lax.fori_loop(0, nq, fire, 0)

        def drain(ci, _):
            _wait(dinv_hbm.at[s_q.at[ci]], ds_b.at[ci], gsem)
            _wait(dinv_hbm.at[d_q.at[ci]], dd_b.at[ci], gsem)
            return 0

        lax.fori_loop(0, nq, drain, 0)

        def cbody(ci, _):
            for g in range(CH // LANES):
                sl = pl.ds(g * LANES, LANES)
                n_q[ci, sl] = -(ds_b[ci, sl] * w_q[ci, sl] * dd_b[ci, sl])
            return 0

        lax.fori_loop(0, nq, cbody, 0)
        pltpu.sync_copy(n_q, out_hbm.at[wid, pl.ds(q * nq, nq)])


def _norm_call(src32, dst32, w32, dinv_flat):
    nq = NCH32 // 5
    kfn = pl.kernel(
        _sc_norm,
        out_type=jax.ShapeDtypeStruct((32, NCH32, CH), jnp.float32),
        mesh=_mesh(),
        compiler_params=_SC_PARAMS,
        scratch_types=[
            pltpu.VMEM((nq, CH), jnp.int32),
            pltpu.VMEM((nq, CH), jnp.int32),
            pltpu.VMEM((nq, CH), jnp.float32),
            pltpu.VMEM((nq, CH), jnp.float32),
            pltpu.VMEM((nq, CH), jnp.float32),
            pltpu.VMEM((nq, CH), jnp.float32),
            pltpu.SemaphoreType.DMA,
        ],
    )
    return kfn(src32, dst32, w32, dinv_flat)


def _sc_scat(x_hbm, src_hbm, dst_hbm, nrm_hbm, out_hbm,
             s_q, d_q, n_q, rb, zb, acc, gsem, ssem):
    c = lax.axis_index("c")
    s = lax.axis_index("s")

    for p in range(2):
        def zbody(i, _):
            zb[i, pl.ds(0, LANES)] = jnp.zeros((LANES,), jnp.float32)
            return 0

        lax.fori_loop(0, ZROWS, zbody, 0)
        for k in range(ROWS_PT // ZROWS):
            pltpu.sync_copy(
                zb, acc.at[pl.ds(s * ROWS_PT + k * ZROWS, ZROWS), :])
        plsc.subcore_barrier()

        xc = x_hbm.at[c, p]

        for q in range(5):
            pltpu.sync_copy(src_hbm.at[s, pl.ds(q * QCH, QCH)], s_q)
            pltpu.sync_copy(dst_hbm.at[s, pl.ds(q * QCH, QCH)], d_q)
            pltpu.sync_copy(nrm_hbm.at[s, pl.ds(q * QCH, QCH)], n_q)

            for k in range(3):
                pltpu.async_copy(xc.at[s_q.at[k]], rb.at[k], gsem.at[k])

            def cbody(c4, _):
                for b in range(4):
                    ci = c4 * 4 + b
                    _wait(xc.at[s_q.at[ci]], rb.at[b], gsem.at[b])

                    def ebody(g, _):
                        nv = n_q[ci, pl.ds(g * LANES, LANES)]
                        for j in range(LANES):
                            e = g * LANES + j
                            nrm = nv[j]
                            v0 = rb[b, e, pl.ds(0, LANES)]
                            rb[b, e, pl.ds(0, LANES)] = v0 * nrm
                        return 0

                    lax.fori_loop(0, CH // LANES, ebody, 0)
                    pltpu.async_copy(rb.at[b], acc.at[d_q.at[ci]],
                                     ssem.at[b], add=True)
                    bn = (b + 3) % 4
                    nxt = ci + 3

                    @pl.when(nxt < QCH)
                    def _():
                        @pl.when(ci >= 1)
                        def _():
                            _wait(rb.at[bn], acc.at[d_q.at[ci]], ssem.at[bn])
                        pltpu.async_copy(xc.at[s_q.at[nxt]], rb.at[bn],
                                         gsem.at[bn])
                return 0

            lax.fori_loop(0, QCH // 4, cbody, 0)
            for b in range(4):
                _wait(rb.at[b], acc.at[d_q.at[QCH - 4 + b]], ssem.at[b])

        plsc.subcore_barrier()
        for k in range(ROWS_PT // ZROWS):
            r0 = s * ROWS_PT + k * ZROWS
            pltpu.sync_copy(acc.at[pl.ds(r0, ZROWS), :], zb)
            pltpu.sync_copy(zb, out_hbm.at[c, p, pl.ds(r0, ZROWS), :])
        plsc.subcore_barrier()


def _scat_call(x4, src16, dst16, nrm16):
    kfn = pl.kernel(
        _sc_scat,
        out_type=jax.ShapeDtypeStruct((NC, 2, N_PAD, LANES), jnp.float32),
        mesh=_mesh(),
        compiler_params=_SC_PARAMS,
        scratch_types=[
            pltpu.VMEM((QCH, CH), jnp.int32),
            pltpu.VMEM((QCH, CH), jnp.int32),
            pltpu.VMEM((QCH, CH), jnp.float32),
            pltpu.VMEM((4, CH, LANES), jnp.float32),
            pltpu.VMEM((ZROWS, LANES), jnp.float32),
            pltpu.VMEM_SHARED((N_PAD, LANES), jnp.float32),
            pltpu.SemaphoreType.DMA((4,)),
            pltpu.SemaphoreType.DMA((4,)),
        ],
    )
    return kfn(x4, src16, dst16, nrm16)


def _mm(a, w):
    return jax.lax.dot_general(a, w, (((1,), (0,)), ((), ())),
                               preferred_element_type=jnp.float32)


def _mm4(ref4, w):
    acc = _mm(ref4[0, 0], w[0:16])
    acc += _mm(ref4[0, 1], w[16:32])
    acc += _mm(ref4[1, 0], w[32:48])
    acc += _mm(ref4[1, 1], w[48:64])
    return acc


def _tc_dinv(deg_ref, out_ref):
    deg = deg_ref[0] + deg_ref[1]
    out_ref[...] = jnp.where(
        deg > 0, lax.rsqrt(jnp.maximum(deg, 1e-12)), 0.0)


def _dinv_call(deg2):
    d = deg2.reshape(NC, N_PAD // 128, 128)
    out = pl.pallas_call(
        _tc_dinv,
        out_shape=jax.ShapeDtypeStruct((N_PAD // 128, 128), jnp.float32),
    )(d)
    return out.reshape(N_PAD)


def _split4(h, out4):
    for k in range(4):
        out4[k // 2, k % 2] = h[:, 16 * k:16 * k + 16]


def _tc_pre(xall_ref, ce_ref, sx_ref, sce_ref, wx0_ref, wx1_ref,
            wce0_ref, wce1_ref, bx_ref, bh_ref, *out_refs):
    xall = xall_ref[...]
    ce = ce_ref[...]

    cce = []
    for g in range(3):
        cce.append(_mm(ce, wce0_ref[g]) + _mm4(sce_ref, wce1_ref[g])
                   + bx_ref[g])

    def A(t, g):
        return _mm(xall, wx0_ref[t, g]) + _mm4(sx_ref, wx1_ref[t, g]) + cce[g]

    k = 0
    for t in range(1, T):
        for g in range(3):
            out_refs[k][...] = A(t, g)
            k += 1
    z0 = jax.nn.sigmoid(A(0, 0) + bh_ref[0])
    h0 = (1.0 - z0) * jnp.tanh(A(0, 2) + bh_ref[2])
    _split4(h0, out_refs[9])


def _pre_call(xall, ce, sx4, sce4, wx0, wx1, wce0, wce1, bx, bh):
    row = lambda i: (i, 0)
    row4 = lambda i: (0, 0, i, 0)
    full = lambda r: pl.BlockSpec(r, lambda i: (0,) * len(r))
    outs = ([jax.ShapeDtypeStruct((N_PAD, EMB), jnp.float32)] * 9
            + [jax.ShapeDtypeStruct((NC, 2, N_PAD, 16), jnp.float32)])
    out_specs = ([pl.BlockSpec((BLK, EMB), row)] * 9
                 + [pl.BlockSpec((NC, 2, BLK, 16), row4)])
    return pl.pallas_call(
        _tc_pre,
        grid=(GRID_N,),
        in_specs=[
            pl.BlockSpec((BLK, EMB), row),
            pl.BlockSpec((BLK, EMB), row),
            pl.BlockSpec((NC, 2, BLK, 16), row4),
            pl.BlockSpec((NC, 2, BLK, 16), row4),
            full((T, 3, EMB, EMB)),
            full((T, 3, EMB, EMB)),
            full((3, EMB, EMB)),
            full((3, EMB, EMB)),
            full((3, EMB)),
            full((3, EMB)),
        ],
        out_specs=out_specs,
        out_shape=outs,
    )(xall, ce, sx4, sce4, wx0, wx1, wce0, wce1, bx, bh)


def _cat4(ref4):
    return jnp.concatenate(
        [ref4[0, 0], ref4[0, 1], ref4[1, 0], ref4[1, 1]], axis=1)


def _tc_zr(h_ref, sh_ref, az_ref, ar_ref, wh_ref, bh_ref, z_ref, hr_ref):
    z = jax.nn.sigmoid(az_ref[...] + _mm4(h_ref, wh_ref[0, 0])
                       + _mm4(sh_ref, wh_ref[0, 1]) + bh_ref[0])
    r = jax.nn.sigmoid(ar_ref[...] + _mm4(h_ref, wh_ref[1, 0])
                       + _mm4(sh_ref, wh_ref[1, 1]) + bh_ref[1])
    z_ref[...] = z
    for k in range(4):
        hr_ref[k // 2, k % 2] = (h_ref[k // 2, k % 2]
                                 * r[:, 16 * k:16 * k + 16])


def _zr_call(h4, sh4, az, ar, wh, bh):
    row = lambda i: (i, 0)
    row4 = lambda i: (0, 0, i, 0)
    full = lambda r: pl.BlockSpec(r, lambda i: (0,) * len(r))
    return pl.pallas_call(
        _tc_zr,
        grid=(GRID_N,),
        in_specs=[
            pl.BlockSpec((NC, 2, BLK, 16), row4),
            pl.BlockSpec((NC, 2, BLK, 16), row4),
            pl.BlockSpec((BLK, EMB), row),
            pl.BlockSpec((BLK, EMB), row),
            full((3, 2, EMB, EMB)),
            full((3, EMB)),
        ],
        out_specs=[pl.BlockSpec((BLK, EMB), row),
                   pl.BlockSpec((NC, 2, BLK, 16), row4)],
        out_shape=[jax.ShapeDtypeStruct((N_PAD, EMB), jnp.float32),
                   jax.ShapeDtypeStruct((NC, 2, N_PAD, 16), jnp.float32)],
    )(h4, sh4, az, ar, wh, bh)


def _tc_ht(last, z_ref, h_ref, hr_ref, shr_ref, ah_ref, wh_ref, bh_ref,
           w1_ref, b1_ref, w2_ref, b2_ref, out_ref):
    htl = jnp.tanh(ah_ref[...] + _mm4(hr_ref, wh_ref[2, 0])
                   + _mm4(shr_ref, wh_ref[2, 1]) + bh_ref[2])
    z = z_ref[...]
    hn = z * _cat4(h_ref) + (1.0 - z) * htl
    if last:
        hid = jax.nn.relu(_mm(hn, w1_ref[...]) + b1_ref[...])
        out_ref[...] = _mm(hid, w2_ref[...]) + b2_ref[...]
    else:
        _split4(hn, out_ref)


def _ht_call(last, z, h4, hr4, shr4, ah, wh, bh, w1, b1, w2p, b2p):
    row = lambda i: (i, 0)
    row4 = lambda i: (0, 0, i, 0)
    full = lambda r: pl.BlockSpec(r, lambda i: (0,) * len(r))
    if last:
        out_spec = pl.BlockSpec((BLK, 8), row)
        out_shape = jax.ShapeDtypeStruct((N_PAD, 8), jnp.float32)
    else:
        out_spec = pl.BlockSpec((NC, 2, BLK, 16), row4)
        out_shape = jax.ShapeDtypeStruct((NC, 2, N_PAD, 16), jnp.float32)
    return pl.pallas_call(
        functools.partial(_tc_ht, last),
        grid=(GRID_N,),
        in_specs=[
            pl.BlockSpec((BLK, EMB), row),
            pl.BlockSpec((NC, 2, BLK, 16), row4),
            pl.BlockSpec((NC, 2, BLK, 16), row4),
            pl.BlockSpec((NC, 2, BLK, 16), row4),
            pl.BlockSpec((BLK, EMB), row),
            full((3, 2, EMB, EMB)),
            full((3, EMB)),
            full((EMB, EMB // 2)),
            full((EMB // 2,)),
            full((EMB // 2, 8)),
            full((8,)),
        ],
        out_specs=out_spec,
        out_shape=out_shape,
    )(z, h4, hr4, shr4, ah, wh, bh, w1, b1, w2p, b2p)


def kernel(x, edge_index, edge_weight, county_emb, Wx, bx, Wh, bh,
           W1, b1, W2, b2):
    f32 = jnp.float32
    pad_e = E_PAD - E
    srcp = jnp.pad(edge_index[0], (0, pad_e))
    dstp = jnp.pad(edge_index[1], (0, pad_e))
    wp = jnp.pad(edge_weight, (0, pad_e))
    src32 = srcp.reshape(32, NCH32, CH)
    dst32 = dstp.reshape(32, NCH32, CH)
    w32 = wp.reshape(32, NCH32, CH)
    src16 = srcp.reshape(16, NCH16, CH)
    dst16 = dstp.reshape(16, NCH16, CH)

    xall = jnp.pad(jnp.moveaxis(x, 0, 1).reshape(N, T * F_DIM),
                   ((0, N_PAD - N), (0, EMB - T * F_DIM)))
    ce = jnp.pad(county_emb, ((0, N_PAD - N), (0, 0)))

    def to4(a):
        return jnp.stack([a[:, 16 * k:16 * k + 16]
                          for k in range(4)]).reshape(2, 2, N_PAD, 16)

    xall4 = to4(xall)
    ce4 = to4(ce)

    WX0 = jnp.zeros((T, 3, EMB, EMB), f32)
    WX1 = jnp.zeros((T, 3, EMB, EMB), f32)
    for t in range(T):
        for g in range(3):
            WX0 = WX0.at[t, g, 15 * t:15 * t + 15].set(Wx[g, 0, :F_DIM])
            WX1 = WX1.at[t, g, 15 * t:15 * t + 15].set(Wx[g, 1, :F_DIM])
    WCE0 = Wx[:, 0, F_DIM:, :]
    WCE1 = Wx[:, 1, F_DIM:, :]
    W2p = jnp.pad(W2, ((0, 0), (0, 7)))
    b2p = jnp.pad(b2, (0, 7))

    deg2 = _deg_call(src32, w32).reshape(NC, N_PAD)
    dinv = _dinv_call(deg2)
    nrm32 = _norm_call(src32, dst32, w32, dinv)
    nrm16 = nrm32.reshape(16, NCH16, CH)

    sx4 = _scat_call(xall4, src16, dst16, nrm16)
    sce4 = _scat_call(ce4, src16, dst16, nrm16)

    pre = _pre_call(xall, ce, sx4, sce4, WX0, WX1, WCE0, WCE1, bx, bh)
    As = pre[:9]
    h4 = pre[9]

    for t in range(1, T):
        az, ar, ah = As[3 * (t - 1)], As[3 * (t - 1) + 1], As[3 * (t - 1) + 2]
        sh4 = _scat_call(h4, src16, dst16, nrm16)
        z, hr4 = _zr_call(h4, sh4, az, ar, Wh, bh)
        shr4 = _scat_call(hr4, src16, dst16, nrm16)
        out = _ht_call(t == T - 1, z, h4, hr4, shr4, ah, Wh, bh,
                       W1, b1, W2p, b2p)
        h4 = out
    return out[:N, 0]
